# Initial kernel scaffold; baseline (speedup 1.0000x reference)
import jax
import jax.numpy as jnp
from jax.experimental import pallas as pl


def kernel(token_ids, weights):
    raise NotImplementedError("write your pallas kernel here")


# speedup vs baseline: 1.0940x; 1.0940x over previous
import functools

import jax
import jax.numpy as jnp
from jax import lax
from jax.experimental import pallas as pl
from jax.experimental.pallas import tpu as pltpu
from jax.experimental.pallas import tpu_sc as plsc

NUM_CORES = 2
NUM_SUBCORES = 16
NUM_WORKERS = NUM_CORES * NUM_SUBCORES


def _make_gather(total_rows: int, dim: int, chunk: int):
    assert total_rows % (NUM_WORKERS * chunk) == 0
    rows_per_worker = total_rows // NUM_WORKERS
    n_chunks = rows_per_worker // chunk
    mesh = plsc.VectorSubcoreMesh(
        core_axis_name="c", subcore_axis_name="s",
        num_cores=NUM_CORES, num_subcores=NUM_SUBCORES)

    @functools.partial(
        pl.kernel,
        out_type=jax.ShapeDtypeStruct((total_rows, dim), jnp.float32),
        mesh=mesh,
        scratch_types=[
            pltpu.VMEM((chunk,), jnp.int32),
            pltpu.VMEM((chunk, dim), jnp.float32),
            pltpu.SemaphoreType.DMA,
        ],
        compiler_params=pltpu.CompilerParams(use_tc_tiling_on_sc=False),
    )
    def gather_kernel(tok_hbm, table_hbm, out_hbm, idx_v, rows_v, sem):
        wid = lax.axis_index("s") * NUM_CORES + lax.axis_index("c")
        base = wid * rows_per_worker

        def body(i, carry):
            start = base + i * chunk
            pltpu.sync_copy(tok_hbm.at[pl.ds(start, chunk)], idx_v)
            pltpu.async_copy(table_hbm.at[idx_v], rows_v, sem).wait()
            pltpu.sync_copy(rows_v, out_hbm.at[pl.ds(start, chunk)])
            return carry

        lax.fori_loop(0, n_chunks, body, 0)

    return gather_kernel


def kernel(token_ids, weights):
    batch, hist = token_ids.shape
    dim = weights.shape[1]
    total = batch * hist
    flat_ids = token_ids.reshape(total)
    out_flat = _make_gather(total, dim, chunk=1024)(flat_ids, weights)
    return out_flat.reshape(batch, hist, dim)


# speedup vs baseline: 1.1134x; 1.0177x over previous
import functools

import jax
import jax.numpy as jnp
from jax import lax
from jax.experimental import pallas as pl
from jax.experimental.pallas import tpu as pltpu
from jax.experimental.pallas import tpu_sc as plsc

NUM_CORES = 2
NUM_SUBCORES = 16
NUM_WORKERS = NUM_CORES * NUM_SUBCORES
NBUF = 4


def _make_gather(total_rows: int, dim: int, chunk: int):
    assert total_rows % (NUM_WORKERS * chunk * NBUF) == 0
    rows_per_worker = total_rows // NUM_WORKERS
    n_chunks = rows_per_worker // chunk
    n_groups = n_chunks // NBUF
    mesh = plsc.VectorSubcoreMesh(
        core_axis_name="c", subcore_axis_name="s",
        num_cores=NUM_CORES, num_subcores=NUM_SUBCORES)

    @functools.partial(
        pl.kernel,
        out_type=jax.ShapeDtypeStruct((total_rows, dim), jnp.float32),
        mesh=mesh,
        scratch_types=[
            pltpu.VMEM((NBUF, chunk), jnp.int32),
            pltpu.VMEM((NBUF, chunk, dim), jnp.float32),
            pltpu.SemaphoreType.DMA((NBUF,)),
            pltpu.SemaphoreType.DMA((NBUF,)),
        ],
        compiler_params=pltpu.CompilerParams(use_tc_tiling_on_sc=False),
    )
    def gather_kernel(tok_hbm, table_hbm, out_hbm, idx_v, rows_v, g_sem, w_sem):
        wid = lax.axis_index("s") * NUM_CORES + lax.axis_index("c")
        base = wid * rows_per_worker

        def fire_gather(i, b):
            pltpu.sync_copy(tok_hbm.at[pl.ds(base + i * chunk, chunk)],
                            idx_v.at[b])
            pltpu.make_async_copy(table_hbm.at[idx_v.at[b]], rows_v.at[b],
                                  g_sem.at[b]).start()

        def drain_gather_fire_write(i, b):
            pltpu.make_async_copy(table_hbm.at[idx_v.at[b]], rows_v.at[b],
                                  g_sem.at[b]).wait()
            pltpu.make_async_copy(rows_v.at[b],
                                  out_hbm.at[pl.ds(base + i * chunk, chunk)],
                                  w_sem.at[b]).start()

        def wait_write(b):
            pltpu.make_async_copy(rows_v.at[b], out_hbm.at[pl.ds(0, chunk)],
                                  w_sem.at[b]).wait()

        fire_gather(0, 0)
        for b in range(1, NBUF):
            fire_gather(b, b)
            drain_gather_fire_write(b - 1, b - 1)

        def group_body(g, carry):
            for b in range(NBUF):
                i = g * NBUF + b
                wait_write(b)
                fire_gather(i, b)
                bp = (b - 1) % NBUF
                drain_gather_fire_write(i - 1, bp)
            return carry

        lax.fori_loop(1, n_groups, group_body, 0)

        drain_gather_fire_write(n_chunks - 1, NBUF - 1)
        for b in range(NBUF):
            wait_write(b)

    return gather_kernel


def kernel(token_ids, weights):
    batch, hist = token_ids.shape
    dim = weights.shape[1]
    total = batch * hist
    flat_ids = token_ids.reshape(total)
    out_flat = _make_gather(total, dim, chunk=800)(flat_ids, weights)
    return out_flat.reshape(batch, hist, dim)


# speedup vs baseline: 1.8081x; 1.6239x over previous
import functools

import jax
import jax.numpy as jnp
from jax import lax
from jax.experimental import pallas as pl
from jax.experimental.pallas import tpu as pltpu
from jax.experimental.pallas import tpu_sc as plsc

NUM_CORES = 2
NUM_SUBCORES = 16
NUM_WORKERS = NUM_CORES * NUM_SUBCORES
NBUF = 4


def _make_gather(total_rows: int, hist: int, dim: int, chunk: int):
    assert total_rows % (NUM_WORKERS * chunk * NBUF) == 0 and chunk % hist == 0
    rows_per_worker = total_rows // NUM_WORKERS
    n_chunks = rows_per_worker // chunk
    n_groups = n_chunks // NBUF
    bpc = chunk // hist
    mesh = plsc.VectorSubcoreMesh(
        core_axis_name="c", subcore_axis_name="s",
        num_cores=NUM_CORES, num_subcores=NUM_SUBCORES)

    @functools.partial(
        pl.kernel,
        out_type=jax.ShapeDtypeStruct((total_rows // hist, hist, dim),
                                      jnp.float32),
        mesh=mesh,
        scratch_types=[
            pltpu.VMEM((NBUF, chunk), jnp.int32),
            pltpu.VMEM((NBUF, chunk, dim), jnp.float32),
            pltpu.SemaphoreType.DMA((NBUF,)),
            pltpu.SemaphoreType.DMA((NBUF,)),
        ],
        compiler_params=pltpu.CompilerParams(use_tc_tiling_on_sc=False),
    )
    def gather_kernel(tok_hbm, table_hbm, out_hbm, idx_v, rows_v, g_sem, w_sem):
        wid = lax.axis_index("s") * NUM_CORES + lax.axis_index("c")
        base = wid * rows_per_worker

        def fire_gather(i, b):
            pltpu.sync_copy(tok_hbm.at[pl.ds(base + i * chunk, chunk)],
                            idx_v.at[b])
            pltpu.make_async_copy(table_hbm.at[idx_v.at[b]], rows_v.at[b],
                                  g_sem.at[b]).start()

        def drain_gather_fire_write(i, b):
            pltpu.make_async_copy(table_hbm.at[idx_v.at[b]], rows_v.at[b],
                                  g_sem.at[b]).wait()
            bpos = (base + i * chunk) // hist
            for k in range(bpc):
                pltpu.make_async_copy(rows_v.at[b].at[pl.ds(k * hist, hist)],
                                      out_hbm.at[bpos + k],
                                      w_sem.at[b]).start()

        def wait_write(b):
            for k in range(bpc):
                pltpu.make_async_copy(rows_v.at[b].at[pl.ds(k * hist, hist)],
                                      out_hbm.at[0],
                                      w_sem.at[b]).wait()

        fire_gather(0, 0)
        for b in range(1, NBUF):
            fire_gather(b, b)
            drain_gather_fire_write(b - 1, b - 1)

        def group_body(g, carry):
            for b in range(NBUF):
                i = g * NBUF + b
                wait_write(b)
                fire_gather(i, b)
                bp = (b - 1) % NBUF
                drain_gather_fire_write(i - 1, bp)
            return carry

        lax.fori_loop(1, n_groups, group_body, 0)

        drain_gather_fire_write(n_chunks - 1, NBUF - 1)
        for b in range(NBUF):
            wait_write(b)

    return gather_kernel


def kernel(token_ids, weights):
    batch, hist = token_ids.shape
    dim = weights.shape[1]
    total = batch * hist
    flat_ids = token_ids.reshape(total)
    return _make_gather(total, hist, dim, chunk=800)(flat_ids, weights)


# speedup vs baseline: 1.8297x; 1.0119x over previous
import functools

import jax
import jax.numpy as jnp
from jax import lax
from jax.experimental import pallas as pl
from jax.experimental.pallas import tpu as pltpu
from jax.experimental.pallas import tpu_sc as plsc

NUM_CORES = 2
NUM_SUBCORES = 16
NUM_WORKERS = NUM_CORES * NUM_SUBCORES
NBUF = 4


def _make_gather(total_rows: int, hist: int, dim: int, chunk: int):
    assert total_rows % (NUM_WORKERS * chunk * NBUF) == 0 and chunk % hist == 0
    rows_per_worker = total_rows // NUM_WORKERS
    n_chunks = rows_per_worker // chunk
    n_groups = n_chunks // NBUF
    bpc = chunk // hist
    mesh = plsc.VectorSubcoreMesh(
        core_axis_name="c", subcore_axis_name="s",
        num_cores=NUM_CORES, num_subcores=NUM_SUBCORES)

    @functools.partial(
        pl.kernel,
        out_type=jax.ShapeDtypeStruct((total_rows // hist, hist, dim),
                                      jnp.float32),
        mesh=mesh,
        scratch_types=[
            pltpu.VMEM((NBUF, chunk), jnp.int32),
            pltpu.VMEM((NBUF, chunk, dim), jnp.float32),
            pltpu.SemaphoreType.DMA((NBUF,)),
            pltpu.SemaphoreType.DMA((NBUF,)),
        ],
        compiler_params=pltpu.CompilerParams(use_tc_tiling_on_sc=False),
    )
    def gather_kernel(tok_hbm, table_hbm, out_hbm, idx_v, rows_v, g_sem, w_sem):
        wid = lax.axis_index("s") * NUM_CORES + lax.axis_index("c")
        base = wid * rows_per_worker

        def fire_gather(i, b):
            pltpu.sync_copy(tok_hbm.at[pl.ds(base + i * chunk, chunk)],
                            idx_v.at[b])
            pltpu.make_async_copy(table_hbm.at[idx_v.at[b]], rows_v.at[b],
                                  g_sem.at[b]).start()

        def drain_gather_fire_write(i, b):
            pltpu.make_async_copy(table_hbm.at[idx_v.at[b]], rows_v.at[b],
                                  g_sem.at[b]).wait()
            bpos = (base + i * chunk) // hist
            for k in range(bpc):
                pltpu.make_async_copy(rows_v.at[b].at[pl.ds(k * hist, hist)],
                                      out_hbm.at[bpos + k],
                                      w_sem.at[b]).start()

        def wait_write(b):
            for k in range(bpc):
                pltpu.make_async_copy(rows_v.at[b].at[pl.ds(k * hist, hist)],
                                      out_hbm.at[0],
                                      w_sem.at[b]).wait()

        fire_gather(0, 0)
        for b in range(1, NBUF):
            fire_gather(b, b)
            drain_gather_fire_write(b - 1, b - 1)

        def group_body(g, carry):
            for b in range(NBUF):
                i = g * NBUF + b
                wait_write(b)
                fire_gather(i, b)
                bp = (b - 1) % NBUF
                drain_gather_fire_write(i - 1, bp)
            return carry

        lax.fori_loop(1, n_groups, group_body, 0)

        drain_gather_fire_write(n_chunks - 1, NBUF - 1)
        for b in range(NBUF):
            wait_write(b)

    return gather_kernel


def kernel(token_ids, weights):
    batch, hist = token_ids.shape
    nemb, dim = weights.shape
    total = batch * hist
    lanes = 128
    mult = lanes // dim
    wp = jnp.pad(weights, ((0, 0), (0, lanes - dim))).reshape(nemb * mult, dim)
    flat_ids = token_ids.reshape(total) * mult
    return _make_gather(total, hist, dim, chunk=800)(flat_ids, wp)
